# Initial kernel scaffold; baseline (speedup 1.0000x reference)
import jax
import jax.numpy as jnp
from jax.experimental import pallas as pl


def kernel(x, moving_xyz, weights0, weights1, weights2, idx0, idx1, idx2, rot_mat_01, rot_mat_12, rot_mat_02, rot_mat_20, index_double_02, index_double_12, index_double_01, index_triple_computed):
    raise NotImplementedError("write your pallas kernel here")


# speedup vs baseline: 1.6396x; 1.6396x over previous
import functools

import jax
import jax.numpy as jnp
from jax import lax
from jax.experimental import pallas as pl
from jax.experimental.pallas import tpu as pltpu
from jax.experimental.pallas import tpu_sc as plsc

VERTS_ = [40962, 10242, 2562, 642, 162]
CHS_ = [2, 64, 128, 256, 512, 1024]
NCORES, NSUB = 2, 16
NW = NCORES * NSUB


def _ru(a, b):
    return (a + b - 1) // b * b


MPS_ = [_ru(n, 256) for n in VERTS_]


@functools.lru_cache(maxsize=None)
def _sc_gather_fn(R, D, Mpad, B):
    chunk = Mpad // NW
    T = chunk // B
    mesh = plsc.VectorSubcoreMesh(
        core_axis_name="c", subcore_axis_name="s",
        num_cores=NCORES, num_subcores=NSUB)

    @functools.partial(
        pl.kernel,
        out_type=jax.ShapeDtypeStruct((Mpad, D), jnp.float32),
        mesh=mesh,
        scratch_types=[
            pltpu.VMEM((B,), jnp.int32),
            pltpu.VMEM((B, D), jnp.float32),
            pltpu.SemaphoreType.DMA,
        ],
    )
    def k(table_hbm, idx_hbm, out_hbm, idx_v, rows_v, sem):
        wid = lax.axis_index("s") * NCORES + lax.axis_index("c")
        base = wid * chunk

        def body(t, carry):
            start = pl.multiple_of(base + t * B, 8)
            pltpu.sync_copy(idx_hbm.at[pl.ds(start, B)], idx_v)
            pltpu.async_copy(table_hbm.at[idx_v], rows_v, sem).wait()
            pltpu.sync_copy(rows_v, out_hbm.at[pl.ds(start, B)])
            return carry

        lax.fori_loop(0, T, body, 0)

    return k


def _sc_gather(table, idx, Mpad, B):
    M = idx.shape[0]
    idxp = jnp.pad(idx, (0, Mpad - M)) if Mpad != M else idx
    return _sc_gather_fn(table.shape[0], table.shape[1], Mpad, B)(table, idxp)


def _gather7(table, no, Mp):
    D = table.shape[1]
    g = _sc_gather(table, no, 7 * Mp, 56)
    return g.reshape(Mp, 7 * D)


@functools.lru_cache(maxsize=None)
def _sc_scatter_fn(Rout, D, Mpad, B):
    chunk = Mpad // NW
    T = chunk // B
    mesh = plsc.VectorSubcoreMesh(
        core_axis_name="c", subcore_axis_name="s",
        num_cores=NCORES, num_subcores=NSUB)

    @functools.partial(
        pl.kernel,
        out_type=jax.ShapeDtypeStruct((Rout, D), jnp.int32),
        mesh=mesh,
        scratch_types=[
            pltpu.VMEM((B,), jnp.int32),
            pltpu.VMEM((B, D), jnp.int32),
            pltpu.SemaphoreType.DMA,
        ],
    )
    def k(src_hbm, idx_hbm, out_hbm, idx_v, rows_v, sem):
        wid = lax.axis_index("s") * NCORES + lax.axis_index("c")
        base = wid * chunk

        def body(t, carry):
            start = pl.multiple_of(base + t * B, 8)
            pltpu.sync_copy(idx_hbm.at[pl.ds(start, B)], idx_v)
            pltpu.sync_copy(src_hbm.at[pl.ds(start, B)], rows_v)
            pltpu.async_copy(rows_v, out_hbm.at[idx_v], sem).wait()
            return carry

        lax.fori_loop(0, T, body, 0)

    return k


def _sc_scatter_rows(src, idx, Rout, pad_row):
    M = idx.shape[0]
    B = 128
    Mpad = _ru(M, NW * B)
    srcp = jnp.pad(src, ((0, Mpad - M), (0, 0)))
    idxp = jnp.pad(idx, (0, Mpad - M), constant_values=pad_row)
    return _sc_scatter_fn(Rout, src.shape[1], Mpad, B)(srcp, idxp)


@functools.lru_cache(maxsize=None)
def _mm_fn(M, K, N, bm, bn, has_bias):
    def body(*refs):
        if has_bias:
            x_ref, w_ref, b_ref, o_ref = refs
        else:
            x_ref, w_ref, o_ref = refs
        acc = jnp.dot(x_ref[...], w_ref[...],
                      preferred_element_type=jnp.float32)
        if has_bias:
            acc = acc + b_ref[...]
        o_ref[...] = acc

    in_specs = [
        pl.BlockSpec((bm, K), lambda i, j: (i, 0)),
        pl.BlockSpec((K, bn), lambda i, j: (0, j)),
    ]
    if has_bias:
        in_specs.append(pl.BlockSpec((1, bn), lambda i, j: (0, j)))
    return pl.pallas_call(
        body,
        grid=(M // bm, N // bn),
        in_specs=in_specs,
        out_specs=pl.BlockSpec((bm, bn), lambda i, j: (i, j)),
        out_shape=jax.ShapeDtypeStruct((M, N), jnp.float32),
    )


def _tc_matmul(x, w, bias=None, keep_pad=False):
    M, K = x.shape
    N = w.shape[1]
    Np = _ru(N, 128)
    if Np != N:
        w = jnp.pad(w, ((0, 0), (0, Np - N)))
        if bias is not None:
            bias = jnp.pad(bias, ((0, 0), (0, Np - N)))
    bn = 512 if Np % 512 == 0 else (256 if Np % 256 == 0 else 128)
    bm = 256
    while K * (bm + bn) * 4 > 12_000_000 and bm > 128:
        bm //= 2
    while K * (bm + bn) * 4 > 12_000_000 and bn > 128:
        bn //= 2
    bm = min(bm, M)
    out = _mm_fn(M, K, Np, bm, bn, bias is not None)(
        *((x, w) if bias is None else (x, w, bias)))
    return out if (keep_pad or Np == N) else out[:, :N]


@functools.lru_cache(maxsize=None)
def _stats_fn(M, c, n, bm):
    def body(y_ref, o_ref):
        i = pl.program_id(0)
        y = y_ref[...]
        rows = lax.broadcasted_iota(jnp.int32, (bm, 1), 0) + i * bm
        ym = jnp.where(rows < n, y, 0.0)
        s = jnp.sum(ym, axis=0, keepdims=True)
        s2 = jnp.sum(ym * ym, axis=0, keepdims=True)
        r8 = lax.broadcasted_iota(jnp.int32, (8, 1), 0)
        upd = jnp.where(r8 == 0, s, 0.0) + jnp.where(r8 == 1, s2, 0.0)

        @pl.when(i == 0)
        def _():
            o_ref[...] = jnp.zeros_like(o_ref)

        o_ref[...] += upd

    return pl.pallas_call(
        body,
        grid=(M // bm,),
        in_specs=[pl.BlockSpec((bm, c), lambda i: (i, 0))],
        out_specs=pl.BlockSpec((8, c), lambda i: (0, 0)),
        out_shape=jax.ShapeDtypeStruct((8, c), jnp.float32),
    )


@functools.lru_cache(maxsize=None)
def _apply_fn(M, c, n, bm):
    inv_n = 1.0 / n

    def body(y_ref, st_ref, gb_ref, o_ref):
        st = st_ref[...]
        mean = st[0:1, :] * inv_n
        var = st[1:2, :] * inv_n - mean * mean
        scale = lax.rsqrt(var + 1e-5) * gb_ref[0:1, :]
        xh = (y_ref[...] - mean) * scale + gb_ref[1:2, :]
        o_ref[...] = jnp.where(xh >= 0, xh, 0.2 * xh)

    return pl.pallas_call(
        body,
        grid=(M // bm,),
        in_specs=[
            pl.BlockSpec((bm, c), lambda i: (i, 0)),
            pl.BlockSpec((8, c), lambda i: (0, 0)),
            pl.BlockSpec((8, c), lambda i: (0, 0)),
        ],
        out_specs=pl.BlockSpec((bm, c), lambda i: (i, 0)),
        out_shape=jax.ShapeDtypeStruct((M, c), jnp.float32),
    )


def _bn_act(y, n, gamma, beta):
    M, cp = y.shape
    c = gamma.shape[0]
    if cp != c:
        gamma = jnp.pad(gamma, (0, cp - c))
        beta = jnp.pad(beta, (0, cp - c))
    gb = jnp.zeros((8, cp), jnp.float32).at[0].set(gamma).at[1].set(beta)
    st = _stats_fn(M, cp, n, 256)(y)
    return _apply_fn(M, cp, n, 256)(y, st, gb)


@functools.lru_cache(maxsize=None)
def _merge_fn(M, bm):
    def body(p0_ref, p1_ref, p2_ref, sid_ref, mats_ref, o_ref):
        p0 = p0_ref[...]
        p1 = p1_ref[...]
        p2 = p2_ref[...]
        sid = sid_ref[:, 0:1]
        acc = jnp.zeros((bm, 8), jnp.float32)
        for s in range(4):
            A = mats_ref[(3 * s) * 8:(3 * s + 1) * 8, :]
            B = mats_ref[(3 * s + 1) * 8:(3 * s + 2) * 8, :]
            C = mats_ref[(3 * s + 2) * 8:(3 * s + 3) * 8, :]
            v = (jnp.dot(p0, A, preferred_element_type=jnp.float32)
                 + jnp.dot(p1, B, preferred_element_type=jnp.float32)
                 + jnp.dot(p2, C, preferred_element_type=jnp.float32))
            acc += jnp.where(sid == s, v, 0.0)
        o_ref[...] = acc

    return pl.pallas_call(
        body,
        grid=(M // bm,),
        in_specs=[
            pl.BlockSpec((bm, 8), lambda i: (i, 0)),
            pl.BlockSpec((bm, 8), lambda i: (i, 0)),
            pl.BlockSpec((bm, 8), lambda i: (i, 0)),
            pl.BlockSpec((bm, 128), lambda i: (i, 0)),
            pl.BlockSpec((96, 8), lambda i: (0, 0)),
        ],
        out_specs=pl.BlockSpec((bm, 8), lambda i: (i, 0)),
        out_shape=jax.ShapeDtypeStruct((M, 8), jnp.float32),
    )


def _prep_conv_w(W, cout, cin_r, cin_p=None):
    cin_p = cin_p or cin_r
    W3 = W.reshape(cout, 7, cin_r)
    if cin_p != cin_r:
        W3 = jnp.pad(W3, ((0, 0), (0, 0), (0, cin_p - cin_r)))
    return jnp.transpose(W3, (1, 2, 0)).reshape(7 * cin_p, cout)


def _pool_mat(c_real, c_pad):
    a = jnp.arange(7 * c_pad)
    k, d = a // c_pad, a % c_pad
    m = k * c_real + d
    valid = d < c_real
    return jnp.where(valid[:, None]
                     & ((m // 7)[:, None] == jnp.arange(c_real)[None, :]),
                     1.0 / 7.0, 0.0)


def _pair_mat(C, Cp):
    a = jnp.arange(Cp)
    return jnp.where((a[:, None] // 2 == jnp.arange(C // 2)[None, :])
                     & (a[:, None] < C), 0.5, 0.0)


def _prep_upconv_w(uw, ub, cin, C, Cp):
    W3 = uw.reshape(7, C, cin)
    b2 = ub.reshape(7, C)
    if Cp != C:
        W3 = jnp.pad(W3, ((0, 0), (0, Cp - C), (0, 0)))
        b2 = jnp.pad(b2, ((0, 0), (0, Cp - C)))
    return (jnp.transpose(W3, (2, 0, 1)).reshape(cin, 7 * Cp),
            b2.reshape(1, 7 * Cp))


def _unet(xp, w, idx):
    acts = [None] * 5
    h = xp
    for i in range(5):
        n = VERTS_[i]
        Mp = MPS_[i]
        cout = CHS_[i + 1]
        if i > 0:
            cprev = CHS_[i]
            g = _gather7(h, idx['neigh'][i - 1][:7 * n], Mp)
            h = _tc_matmul(g, _pool_mat(cprev, h.shape[1]), keep_pad=True)
        cin_r = 2 if i == 0 else CHS_[i]
        g = _gather7(h, idx['neigh'][i], Mp)
        y = _tc_matmul(g, _prep_conv_w(w['dw1'][i], cout, cin_r, h.shape[1]),
                       keep_pad=True)
        a = _bn_act(y, n, w['g1'][i], w['be1'][i])
        g = _gather7(a, idx['neigh'][i], Mp)
        y = _tc_matmul(g, _prep_conv_w(w['dw2'][i], cout, cout, a.shape[1]),
                       keep_pad=True)
        h = _bn_act(y, n, w['g2'][i], w['be2'][i])
        acts[i] = h
    for i in range(4):
        C = CHS_[4 - i]
        Cp = _ru(C, 128)
        nc = VERTS_[4 - i]
        nf = VERTS_[3 - i]
        lev = 3 - i
        Mpf = MPS_[lev]
        Mpc = MPS_[4 - i]
        down = idx['up'][lev][1]
        cin = CHS_[5 - i]
        Wup, bup = _prep_upconv_w(w['uw'][i], w['ub'][i], cin, C, Cp)
        Hup = _tc_matmul(h[:, :cin], Wup, bias=bup, keep_pad=True)
        x1 = Hup[:nc, :C]
        Hr = Hup.reshape(Mpc * 7, Cp)
        Md = _ru(2 * (nf - nc), NW * 128)
        gd = _sc_gather(Hr, down, Md, 128)
        pairs = _tc_matmul(gd, _pair_mat(C, Cp))
        x2 = pairs[:2 * (nf - nc)].reshape(nf - nc, C)
        hcat = jnp.concatenate(
            [jnp.concatenate([x1, x2], axis=0), acts[lev][:nf, :C]], axis=1)
        hcat = jnp.pad(hcat, ((0, Mpf - nf), (0, 0)))
        no = idx['neigh'][lev]
        g = _gather7(hcat, no, Mpf)
        y = _tc_matmul(g, _prep_conv_w(w['c1w'][i], C, 2 * C, hcat.shape[1]),
                       keep_pad=True)
        a = _bn_act(y, nf, w['bg1'][i], w['bb1'][i])
        g = _gather7(a, no, Mpf)
        y = _tc_matmul(g, _prep_conv_w(w['c2w'][i], C, C, a.shape[1]),
                       keep_pad=True)
        h = _bn_act(y, nf, w['bg2'][i], w['bb2'][i])
    outWt = jnp.pad(jnp.transpose(w['outW']) / 50.0,
                    ((0, h.shape[1] - 64), (0, 5)))
    outb = jnp.pad(w['outb'] / 50.0, (0, 5))[None, :]
    return _tc_matmul(h, outWt, bias=outb)


def _p8(m):
    return jnp.pad(m, ((0, 5), (0, 5)))


def kernel(x, moving_xyz, weights0, weights1, weights2, idx0, idx1, idx2,
           rot_mat_01, rot_mat_12, rot_mat_02, rot_mat_20,
           index_double_02, index_double_12, index_double_01,
           index_triple_computed):
    N = x.shape[0]
    Mp0 = MPS_[0]
    xp = jnp.pad(x, ((0, Mp0 - N), (0, 126)))
    phis = [_unet(xp, w, idx) for w, idx in
            ((weights0, idx0), (weights1, idx1), (weights2, idx2))]

    n4 = N // 8
    perm = jnp.concatenate([index_double_02, index_double_12,
                            index_double_01, index_triple_computed])
    sval = jnp.concatenate([
        jnp.zeros((n4,), jnp.int32),
        jnp.full((n4,), 1, jnp.int32),
        jnp.full((n4,), 2, jnp.int32),
        jnp.full((N - 3 * n4,), 3, jnp.int32)])
    svals = jnp.broadcast_to(sval[:, None], (N, 128))
    sid = _sc_scatter_rows(svals, perm, Mp0, Mp0 - 1)

    T20 = jnp.transpose(rot_mat_20)
    T12_20 = jnp.transpose(rot_mat_12) @ T20
    T02_20 = jnp.transpose(rot_mat_02) @ T20
    z3 = jnp.zeros((3, 3), jnp.float32)
    mats = jnp.concatenate([
        _p8(T02_20), _p8(z3), _p8(z3),
        _p8(z3), _p8(T12_20), _p8(z3),
        _p8(jnp.transpose(rot_mat_01) @ T12_20 / 2), _p8(T12_20 / 2), _p8(z3),
        _p8(z3), _p8(T12_20), _p8(z3),
    ], axis=0)

    out8 = _merge_fn(Mp0, 256)(phis[0], phis[1], phis[2], sid, mats)
    return out8[:N, :3]


# speedup vs baseline: 1.6691x; 1.0180x over previous
import functools

import jax
import jax.numpy as jnp
from jax import lax
from jax.experimental import pallas as pl
from jax.experimental.pallas import tpu as pltpu
from jax.experimental.pallas import tpu_sc as plsc

VERTS_ = [40962, 10242, 2562, 642, 162]
CHS_ = [2, 64, 128, 256, 512, 1024]
NCORES, NSUB = 2, 16
NW = NCORES * NSUB


def _ru(a, b):
    return (a + b - 1) // b * b


MPS_ = [_ru(n, 256) for n in VERTS_]


@functools.lru_cache(maxsize=None)
def _sc_gather_fn(R, D, Mpad, B):
    chunk = Mpad // NW
    T = chunk // B
    mesh = plsc.VectorSubcoreMesh(
        core_axis_name="c", subcore_axis_name="s",
        num_cores=NCORES, num_subcores=NSUB)

    @functools.partial(
        pl.kernel,
        out_type=jax.ShapeDtypeStruct((Mpad, D), jnp.float32),
        mesh=mesh,
        scratch_types=[
            pltpu.VMEM((chunk,), jnp.int32),
            pltpu.VMEM((B, D), jnp.float32),
            pltpu.VMEM((B, D), jnp.float32),
            pltpu.SemaphoreType.DMA,
            pltpu.SemaphoreType.DMA,
            pltpu.SemaphoreType.DMA,
            pltpu.SemaphoreType.DMA,
        ],
    )
    def k(table_hbm, idx_hbm, out_hbm, idx_a, r0, r1, g0, g1, w0, w1):
        wid = lax.axis_index("s") * NCORES + lax.axis_index("c")
        base = pl.multiple_of(wid * chunk, 8)
        pltpu.sync_copy(idx_hbm.at[pl.ds(base, chunk)], idx_a)

        def gather(t, rb, sg):
            pltpu.async_copy(
                table_hbm.at[idx_a.at[pl.ds(t * B, B)]], rb, sg)

        def wait_gather(t, rb, sg):
            pltpu.make_async_copy(
                table_hbm.at[idx_a.at[pl.ds(t * B, B)]], rb, sg).wait()

        gather(0, r0, g0)
        if T > 1:
            gather(1, r1, g1)

        def step(t, rb, sg, sw):
            wait_gather(t, rb, sg)
            dst = out_hbm.at[pl.ds(base + t * B, B)]
            pltpu.async_copy(rb, dst, sw)

            @pl.when(t + 2 < T)
            def _():
                pltpu.make_async_copy(rb, dst, sw).wait()
                gather(t + 2, rb, sg)

        def body(t, carry):
            @pl.when(lax.rem(t, 2) == 0)
            def _():
                step(t, r0, g0, w0)

            @pl.when(lax.rem(t, 2) == 1)
            def _():
                step(t, r1, g1, w1)

            return carry

        lax.fori_loop(0, T, body, 0)
        tail = [T - 1] if T == 1 else [T - 2, T - 1]
        for t in tail:
            rb = (r0, r1)[t % 2]
            sw = (w0, w1)[t % 2]
            pltpu.make_async_copy(
                rb, out_hbm.at[pl.ds(base + t * B, B)], sw).wait()

    return k


def _sc_gather(table, idx, Mpad, B):
    M = idx.shape[0]
    idxp = jnp.pad(idx, (0, Mpad - M)) if Mpad != M else idx
    return _sc_gather_fn(table.shape[0], table.shape[1], Mpad, B)(table, idxp)


def _gather7(table, no, Mp):
    D = table.shape[1]
    g = _sc_gather(table, no, 7 * Mp, 56)
    return g.reshape(Mp, 7 * D)


@functools.lru_cache(maxsize=None)
def _sc_scatter_fn(Rout, D, Mpad, B):
    chunk = Mpad // NW
    T = chunk // B
    mesh = plsc.VectorSubcoreMesh(
        core_axis_name="c", subcore_axis_name="s",
        num_cores=NCORES, num_subcores=NSUB)

    @functools.partial(
        pl.kernel,
        out_type=jax.ShapeDtypeStruct((Rout, D), jnp.int32),
        mesh=mesh,
        scratch_types=[
            pltpu.VMEM((B,), jnp.int32),
            pltpu.VMEM((B, D), jnp.int32),
            pltpu.SemaphoreType.DMA,
        ],
    )
    def k(src_hbm, idx_hbm, out_hbm, idx_v, rows_v, sem):
        wid = lax.axis_index("s") * NCORES + lax.axis_index("c")
        base = wid * chunk

        def body(t, carry):
            start = pl.multiple_of(base + t * B, 8)
            pltpu.sync_copy(idx_hbm.at[pl.ds(start, B)], idx_v)
            pltpu.sync_copy(src_hbm.at[pl.ds(start, B)], rows_v)
            pltpu.async_copy(rows_v, out_hbm.at[idx_v], sem).wait()
            return carry

        lax.fori_loop(0, T, body, 0)

    return k


def _sc_scatter_rows(src, idx, Rout, pad_row):
    M = idx.shape[0]
    B = 128
    Mpad = _ru(M, NW * B)
    srcp = jnp.pad(src, ((0, Mpad - M), (0, 0)))
    idxp = jnp.pad(idx, (0, Mpad - M), constant_values=pad_row)
    return _sc_scatter_fn(Rout, src.shape[1], Mpad, B)(srcp, idxp)


@functools.lru_cache(maxsize=None)
def _mm_fn(M, K, N, bm, bn, has_bias):
    def body(*refs):
        if has_bias:
            x_ref, w_ref, b_ref, o_ref = refs
        else:
            x_ref, w_ref, o_ref = refs
        acc = jnp.dot(x_ref[...], w_ref[...],
                      preferred_element_type=jnp.float32)
        if has_bias:
            acc = acc + b_ref[...]
        o_ref[...] = acc

    in_specs = [
        pl.BlockSpec((bm, K), lambda i, j: (i, 0)),
        pl.BlockSpec((K, bn), lambda i, j: (0, j)),
    ]
    if has_bias:
        in_specs.append(pl.BlockSpec((1, bn), lambda i, j: (0, j)))
    return pl.pallas_call(
        body,
        grid=(M // bm, N // bn),
        in_specs=in_specs,
        out_specs=pl.BlockSpec((bm, bn), lambda i, j: (i, j)),
        out_shape=jax.ShapeDtypeStruct((M, N), jnp.float32),
    )


def _tc_matmul(x, w, bias=None, keep_pad=False):
    M, K = x.shape
    N = w.shape[1]
    Np = _ru(N, 128)
    if Np != N:
        w = jnp.pad(w, ((0, 0), (0, Np - N)))
        if bias is not None:
            bias = jnp.pad(bias, ((0, 0), (0, Np - N)))
    bn = 512 if Np % 512 == 0 else (256 if Np % 256 == 0 else 128)
    bm = 256
    while K * (bm + bn) * 4 > 12_000_000 and bm > 128:
        bm //= 2
    while K * (bm + bn) * 4 > 12_000_000 and bn > 128:
        bn //= 2
    bm = min(bm, M)
    out = _mm_fn(M, K, Np, bm, bn, bias is not None)(
        *((x, w) if bias is None else (x, w, bias)))
    return out if (keep_pad or Np == N) else out[:, :N]


@functools.lru_cache(maxsize=None)
def _stats_fn(M, c, n, bm):
    def body(y_ref, o_ref):
        i = pl.program_id(0)
        y = y_ref[...]
        rows = lax.broadcasted_iota(jnp.int32, (bm, 1), 0) + i * bm
        ym = jnp.where(rows < n, y, 0.0)
        s = jnp.sum(ym, axis=0, keepdims=True)
        s2 = jnp.sum(ym * ym, axis=0, keepdims=True)
        r8 = lax.broadcasted_iota(jnp.int32, (8, 1), 0)
        upd = jnp.where(r8 == 0, s, 0.0) + jnp.where(r8 == 1, s2, 0.0)

        @pl.when(i == 0)
        def _():
            o_ref[...] = jnp.zeros_like(o_ref)

        o_ref[...] += upd

    return pl.pallas_call(
        body,
        grid=(M // bm,),
        in_specs=[pl.BlockSpec((bm, c), lambda i: (i, 0))],
        out_specs=pl.BlockSpec((8, c), lambda i: (0, 0)),
        out_shape=jax.ShapeDtypeStruct((8, c), jnp.float32),
    )


@functools.lru_cache(maxsize=None)
def _apply_fn(M, c, n, bm):
    inv_n = 1.0 / n

    def body(y_ref, st_ref, gb_ref, o_ref):
        st = st_ref[...]
        mean = st[0:1, :] * inv_n
        var = st[1:2, :] * inv_n - mean * mean
        scale = lax.rsqrt(var + 1e-5) * gb_ref[0:1, :]
        xh = (y_ref[...] - mean) * scale + gb_ref[1:2, :]
        o_ref[...] = jnp.where(xh >= 0, xh, 0.2 * xh)

    return pl.pallas_call(
        body,
        grid=(M // bm,),
        in_specs=[
            pl.BlockSpec((bm, c), lambda i: (i, 0)),
            pl.BlockSpec((8, c), lambda i: (0, 0)),
            pl.BlockSpec((8, c), lambda i: (0, 0)),
        ],
        out_specs=pl.BlockSpec((bm, c), lambda i: (i, 0)),
        out_shape=jax.ShapeDtypeStruct((M, c), jnp.float32),
    )


def _bn_act(y, n, gamma, beta):
    M, cp = y.shape
    c = gamma.shape[0]
    if cp != c:
        gamma = jnp.pad(gamma, (0, cp - c))
        beta = jnp.pad(beta, (0, cp - c))
    gb = jnp.zeros((8, cp), jnp.float32).at[0].set(gamma).at[1].set(beta)
    st = _stats_fn(M, cp, n, 256)(y)
    return _apply_fn(M, cp, n, 256)(y, st, gb)


@functools.lru_cache(maxsize=None)
def _merge_fn(M, bm):
    def body(p0_ref, p1_ref, p2_ref, sid_ref, mats_ref, o_ref):
        p0 = p0_ref[...]
        p1 = p1_ref[...]
        p2 = p2_ref[...]
        sid = sid_ref[:, 0:1]
        acc = jnp.zeros((bm, 8), jnp.float32)
        for s in range(4):
            A = mats_ref[(3 * s) * 8:(3 * s + 1) * 8, :]
            B = mats_ref[(3 * s + 1) * 8:(3 * s + 2) * 8, :]
            C = mats_ref[(3 * s + 2) * 8:(3 * s + 3) * 8, :]
            v = (jnp.dot(p0, A, preferred_element_type=jnp.float32)
                 + jnp.dot(p1, B, preferred_element_type=jnp.float32)
                 + jnp.dot(p2, C, preferred_element_type=jnp.float32))
            acc += jnp.where(sid == s, v, 0.0)
        o_ref[...] = acc

    return pl.pallas_call(
        body,
        grid=(M // bm,),
        in_specs=[
            pl.BlockSpec((bm, 8), lambda i: (i, 0)),
            pl.BlockSpec((bm, 8), lambda i: (i, 0)),
            pl.BlockSpec((bm, 8), lambda i: (i, 0)),
            pl.BlockSpec((bm, 128), lambda i: (i, 0)),
            pl.BlockSpec((96, 8), lambda i: (0, 0)),
        ],
        out_specs=pl.BlockSpec((bm, 8), lambda i: (i, 0)),
        out_shape=jax.ShapeDtypeStruct((M, 8), jnp.float32),
    )


def _prep_conv_w(W, cout, cin_r, cin_p=None):
    cin_p = cin_p or cin_r
    W3 = W.reshape(cout, 7, cin_r)
    if cin_p != cin_r:
        W3 = jnp.pad(W3, ((0, 0), (0, 0), (0, cin_p - cin_r)))
    return jnp.transpose(W3, (1, 2, 0)).reshape(7 * cin_p, cout)


def _pool_mat(c_real, c_pad):
    a = jnp.arange(7 * c_pad)
    k, d = a // c_pad, a % c_pad
    m = k * c_real + d
    valid = d < c_real
    return jnp.where(valid[:, None]
                     & ((m // 7)[:, None] == jnp.arange(c_real)[None, :]),
                     1.0 / 7.0, 0.0)


def _pair_mat(C, Cp):
    a = jnp.arange(Cp)
    return jnp.where((a[:, None] // 2 == jnp.arange(C // 2)[None, :])
                     & (a[:, None] < C), 0.5, 0.0)


def _prep_upconv_w(uw, ub, cin, C, Cp):
    W3 = uw.reshape(7, C, cin)
    b2 = ub.reshape(7, C)
    if Cp != C:
        W3 = jnp.pad(W3, ((0, 0), (0, Cp - C), (0, 0)))
        b2 = jnp.pad(b2, ((0, 0), (0, Cp - C)))
    return (jnp.transpose(W3, (2, 0, 1)).reshape(cin, 7 * Cp),
            b2.reshape(1, 7 * Cp))


def _unet(xp, w, idx):
    acts = [None] * 5
    h = xp
    for i in range(5):
        n = VERTS_[i]
        Mp = MPS_[i]
        cout = CHS_[i + 1]
        if i > 0:
            cprev = CHS_[i]
            g = _gather7(h, idx['neigh'][i - 1][:7 * n], Mp)
            h = _tc_matmul(g, _pool_mat(cprev, h.shape[1]), keep_pad=True)
        cin_r = 2 if i == 0 else CHS_[i]
        g = _gather7(h, idx['neigh'][i], Mp)
        y = _tc_matmul(g, _prep_conv_w(w['dw1'][i], cout, cin_r, h.shape[1]),
                       keep_pad=True)
        a = _bn_act(y, n, w['g1'][i], w['be1'][i])
        g = _gather7(a, idx['neigh'][i], Mp)
        y = _tc_matmul(g, _prep_conv_w(w['dw2'][i], cout, cout, a.shape[1]),
                       keep_pad=True)
        h = _bn_act(y, n, w['g2'][i], w['be2'][i])
        acts[i] = h
    for i in range(4):
        C = CHS_[4 - i]
        Cp = _ru(C, 128)
        nc = VERTS_[4 - i]
        nf = VERTS_[3 - i]
        lev = 3 - i
        Mpf = MPS_[lev]
        Mpc = MPS_[4 - i]
        down = idx['up'][lev][1]
        cin = CHS_[5 - i]
        Wup, bup = _prep_upconv_w(w['uw'][i], w['ub'][i], cin, C, Cp)
        Hup = _tc_matmul(h[:, :cin], Wup, bias=bup, keep_pad=True)
        x1 = Hup[:nc, :C]
        Hr = Hup.reshape(Mpc * 7, Cp)
        Bd = 64 if Cp >= 512 else 128
        Md = _ru(2 * (nf - nc), NW * Bd)
        gd = _sc_gather(Hr, down, Md, Bd)
        pairs = _tc_matmul(gd, _pair_mat(C, Cp))
        x2 = pairs[:2 * (nf - nc)].reshape(nf - nc, C)
        hcat = jnp.concatenate(
            [jnp.concatenate([x1, x2], axis=0), acts[lev][:nf, :C]], axis=1)
        hcat = jnp.pad(hcat, ((0, Mpf - nf), (0, 0)))
        no = idx['neigh'][lev]
        g = _gather7(hcat, no, Mpf)
        y = _tc_matmul(g, _prep_conv_w(w['c1w'][i], C, 2 * C, hcat.shape[1]),
                       keep_pad=True)
        a = _bn_act(y, nf, w['bg1'][i], w['bb1'][i])
        g = _gather7(a, no, Mpf)
        y = _tc_matmul(g, _prep_conv_w(w['c2w'][i], C, C, a.shape[1]),
                       keep_pad=True)
        h = _bn_act(y, nf, w['bg2'][i], w['bb2'][i])
    outWt = jnp.pad(jnp.transpose(w['outW']) / 50.0,
                    ((0, h.shape[1] - 64), (0, 5)))
    outb = jnp.pad(w['outb'] / 50.0, (0, 5))[None, :]
    return _tc_matmul(h, outWt, bias=outb)


def _p8(m):
    return jnp.pad(m, ((0, 5), (0, 5)))


def kernel(x, moving_xyz, weights0, weights1, weights2, idx0, idx1, idx2,
           rot_mat_01, rot_mat_12, rot_mat_02, rot_mat_20,
           index_double_02, index_double_12, index_double_01,
           index_triple_computed):
    N = x.shape[0]
    Mp0 = MPS_[0]
    xp = jnp.pad(x, ((0, Mp0 - N), (0, 126)))
    phis = [_unet(xp, w, idx) for w, idx in
            ((weights0, idx0), (weights1, idx1), (weights2, idx2))]

    n4 = N // 8
    perm = jnp.concatenate([index_double_02, index_double_12,
                            index_double_01, index_triple_computed])
    sval = jnp.concatenate([
        jnp.zeros((n4,), jnp.int32),
        jnp.full((n4,), 1, jnp.int32),
        jnp.full((n4,), 2, jnp.int32),
        jnp.full((N - 3 * n4,), 3, jnp.int32)])
    svals = jnp.broadcast_to(sval[:, None], (N, 128))
    sid = _sc_scatter_rows(svals, perm, Mp0, Mp0 - 1)

    T20 = jnp.transpose(rot_mat_20)
    T12_20 = jnp.transpose(rot_mat_12) @ T20
    T02_20 = jnp.transpose(rot_mat_02) @ T20
    z3 = jnp.zeros((3, 3), jnp.float32)
    mats = jnp.concatenate([
        _p8(T02_20), _p8(z3), _p8(z3),
        _p8(z3), _p8(T12_20), _p8(z3),
        _p8(jnp.transpose(rot_mat_01) @ T12_20 / 2), _p8(T12_20 / 2), _p8(z3),
        _p8(z3), _p8(T12_20), _p8(z3),
    ], axis=0)

    out8 = _merge_fn(Mp0, 256)(phis[0], phis[1], phis[2], sid, mats)
    return out8[:N, :3]
